# baseline (speedup 1.0000x reference)
import jax
import jax.numpy as jnp
from jax import lax
from jax.experimental import pallas as pl
from jax.experimental.pallas import tpu as pltpu
from jax.experimental.pallas import tpu_sc as plsc

_B = 16384
_D = 32
_NF = 26
_NW = 32
_BPW = _B // _NW
_R = 2048


def _sc_gather_body(ul, il, uwt, iwt, uidx, iidx,
                    u_out, i_out, uw_out, iw_out,
                    uidx_v, iidx_v, sem_u, sem_i, sem_uw, sem_iw):
    wid = lax.axis_index("s") * 2 + lax.axis_index("c")
    base = wid * _BPW
    pltpu.sync_copy(uidx.at[pl.ds(base, _BPW)], uidx_v)
    pltpu.sync_copy(iidx.at[pl.ds(base, _BPW)], iidx_v)

    def body(c, carry):
        j0 = c * 16
        uvec = uidx_v[pl.ds(j0, 16)]
        ivec = iidx_v[pl.ds(j0, 16)]
        for k in range(16):
            r = uvec[k]
            q = ivec[k]
            j = j0 + k
            pltpu.async_copy(ul.at[r], u_out.at[base + j], sem_u)
            pltpu.async_copy(il.at[q], i_out.at[base + j], sem_i)
            pltpu.async_copy(uwt.at[r], uw_out.at[base + j], sem_uw)
            pltpu.async_copy(iwt.at[q], iw_out.at[base + j], sem_iw)
        return carry

    lax.fori_loop(0, _BPW // 16, body, 0)
    pltpu.make_async_copy(ul.at[pl.ds(0, _BPW)],
                          u_out.at[pl.ds(base, _BPW)], sem_u).wait()
    pltpu.make_async_copy(il.at[pl.ds(0, _BPW)],
                          i_out.at[pl.ds(base, _BPW)], sem_i).wait()
    pltpu.make_async_copy(uwt.at[pl.ds(0, _BPW)],
                          uw_out.at[pl.ds(base, _BPW)], sem_uw).wait()
    pltpu.make_async_copy(iwt.at[pl.ds(0, _BPW)],
                          iw_out.at[pl.ds(base, _BPW)], sem_iw).wait()


_sc_gather = pl.kernel(
    _sc_gather_body,
    mesh=plsc.VectorSubcoreMesh(core_axis_name="c", subcore_axis_name="s"),
    out_type=[
        jax.ShapeDtypeStruct((_B, _D), jnp.float32),
        jax.ShapeDtypeStruct((_B, _D), jnp.float32),
        jax.ShapeDtypeStruct((_B, 1), jnp.float32),
        jax.ShapeDtypeStruct((_B, 1), jnp.float32),
    ],
    scratch_types=[
        pltpu.VMEM((_BPW,), jnp.int32),
        pltpu.VMEM((_BPW,), jnp.int32),
        pltpu.SemaphoreType.DMA,
        pltpu.SemaphoreType.DMA,
        pltpu.SemaphoreType.DMA,
        pltpu.SemaphoreType.DMA,
    ],
)


def _tc_combine_body(feats_ref, u_ref, i_ref, uw_ref, iw_ref,
                     fl_ref, fw_ref, fb_ref, out_ref):
    f = feats_ref[...]
    u = u_ref[...]
    iv = i_ref[...]
    w = fw_ref[...]
    p = lax.dot_general(u, fl_ref[...], (((1,), (1,)), ((), ())),
                        preferred_element_type=jnp.float32)
    lin = jnp.sum(f * w, axis=1, keepdims=True)
    inter1 = jnp.sum(u * iv, axis=1, keepdims=True)
    inter2 = jnp.sum(p * f, axis=1, keepdims=True)
    out_ref[...] = (lin + fb_ref[0, 0] + uw_ref[...] + iw_ref[...]
                    + inter1 + inter2)


def _tc_combine(feats, u_e, i_e, uw, iw, fl, fw, fb):
    nblk = _B // _R
    return pl.pallas_call(
        _tc_combine_body,
        grid=(nblk,),
        in_specs=[
            pl.BlockSpec((_R, _NF), lambda i: (i, 0)),
            pl.BlockSpec((_R, _D), lambda i: (i, 0)),
            pl.BlockSpec((_R, _D), lambda i: (i, 0)),
            pl.BlockSpec((_R, 1), lambda i: (i, 0)),
            pl.BlockSpec((_R, 1), lambda i: (i, 0)),
            pl.BlockSpec((_NF, _D), lambda i: (0, 0)),
            pl.BlockSpec((1, _NF), lambda i: (0, 0)),
            pl.BlockSpec((1, 1), lambda i: (0, 0)),
        ],
        out_specs=pl.BlockSpec((_R, 1), lambda i: (i, 0)),
        out_shape=jax.ShapeDtypeStruct((_B, 1), jnp.float32),
    )(feats, u_e, i_e, uw, iw, fl, fw, fb)


def kernel(x, user_latent, item_latent, feat_latent, fw_W, fw_b,
           user_weight, item_weight):
    users = x[:, 0].astype(jnp.int32)
    items = x[:, 1].astype(jnp.int32)
    feats = x[:, 2:]
    u_e, i_e, uw, iw = _sc_gather(user_latent, item_latent,
                                  user_weight, item_weight, users, items)
    return _tc_combine(feats, u_e, i_e, uw, iw,
                       feat_latent, fw_W, jnp.reshape(fw_b, (1, 1)))


# speedup vs baseline: 1.9198x; 1.9198x over previous
import jax
import jax.numpy as jnp
from jax import lax
from jax.experimental import pallas as pl
from jax.experimental.pallas import tpu as pltpu
from jax.experimental.pallas import tpu_sc as plsc

_B = 16384
_D = 32
_NF = 26
_NW = 32
_BPW = _B // _NW
_R = 2048


def _gather_rows(table, idx_v, buf, out, out_base, n, sem):
    def body(c, carry):
        j0 = c * 16
        vec = idx_v[pl.ds(j0, 16)]
        for k in range(16):
            r = vec[k]
            pltpu.async_copy(table.at[pl.ds(r, 1)],
                             buf.at[pl.ds(j0 + k, 1)], sem)
        return carry

    lax.fori_loop(0, n // 16, body, 0)
    pltpu.make_async_copy(table.at[pl.ds(0, n)], buf, sem).wait()
    pltpu.sync_copy(buf, out.at[pl.ds(out_base, n)])


def _sc_gather_body(ul, il, uwt, iwt, uidx, iidx,
                    ue_out, ie_out, uw_out, iw_out,
                    uidx_v, iidx_v, ebuf, wbuf, sem):
    wid = lax.axis_index("s") * 2 + lax.axis_index("c")
    base = wid * _BPW
    pltpu.sync_copy(uidx.at[pl.ds(base, _BPW)], uidx_v)
    pltpu.sync_copy(iidx.at[pl.ds(base, _BPW)], iidx_v)
    _gather_rows(ul, uidx_v, ebuf, ue_out, base, _BPW, sem)
    _gather_rows(il, iidx_v, ebuf, ie_out, base, _BPW, sem)
    _gather_rows(uwt, uidx_v.at[pl.ds(0, _BPW // 2)], wbuf,
                 uw_out, base, _BPW // 2, sem)
    _gather_rows(uwt, uidx_v.at[pl.ds(_BPW // 2, _BPW // 2)], wbuf,
                 uw_out, base + _BPW // 2, _BPW // 2, sem)
    _gather_rows(iwt, iidx_v.at[pl.ds(0, _BPW // 2)], wbuf,
                 iw_out, base, _BPW // 2, sem)
    _gather_rows(iwt, iidx_v.at[pl.ds(_BPW // 2, _BPW // 2)], wbuf,
                 iw_out, base + _BPW // 2, _BPW // 2, sem)


_sc_gather = pl.kernel(
    _sc_gather_body,
    mesh=plsc.VectorSubcoreMesh(core_axis_name="c", subcore_axis_name="s"),
    out_type=[
        jax.ShapeDtypeStruct((_B, _D), jnp.float32),
        jax.ShapeDtypeStruct((_B, _D), jnp.float32),
        jax.ShapeDtypeStruct((_B, 1), jnp.float32),
        jax.ShapeDtypeStruct((_B, 1), jnp.float32),
    ],
    scratch_types=[
        pltpu.VMEM((_BPW,), jnp.int32),
        pltpu.VMEM((_BPW,), jnp.int32),
        pltpu.VMEM((_BPW, _D), jnp.float32),
        pltpu.VMEM((_BPW // 2, 1), jnp.float32),
        pltpu.SemaphoreType.DMA,
    ],
)


def _tc_combine_body(feats_ref, u_ref, i_ref, uw_ref, iw_ref,
                     fl_ref, fw_ref, fb_ref, out_ref):
    f = feats_ref[...]
    u = u_ref[...]
    iv = i_ref[...]
    w = fw_ref[...]
    p = lax.dot_general(u, fl_ref[...], (((1,), (1,)), ((), ())),
                        preferred_element_type=jnp.float32)
    lin = jnp.sum(f * w, axis=1, keepdims=True)
    inter1 = jnp.sum(u * iv, axis=1, keepdims=True)
    inter2 = jnp.sum(p * f, axis=1, keepdims=True)
    out_ref[...] = (lin + fb_ref[0, 0] + uw_ref[...] + iw_ref[...]
                    + inter1 + inter2)


def _tc_combine(feats, u_e, i_e, uw, iw, fl, fw, fb):
    nblk = _B // _R
    return pl.pallas_call(
        _tc_combine_body,
        grid=(nblk,),
        in_specs=[
            pl.BlockSpec((_R, _NF), lambda i: (i, 0)),
            pl.BlockSpec((_R, _D), lambda i: (i, 0)),
            pl.BlockSpec((_R, _D), lambda i: (i, 0)),
            pl.BlockSpec((_R, 1), lambda i: (i, 0)),
            pl.BlockSpec((_R, 1), lambda i: (i, 0)),
            pl.BlockSpec((_NF, _D), lambda i: (0, 0)),
            pl.BlockSpec((1, _NF), lambda i: (0, 0)),
            pl.BlockSpec((1, 1), lambda i: (0, 0)),
        ],
        out_specs=pl.BlockSpec((_R, 1), lambda i: (i, 0)),
        out_shape=jax.ShapeDtypeStruct((_B, 1), jnp.float32),
    )(feats, u_e, i_e, uw, iw, fl, fw, fb)


def kernel(x, user_latent, item_latent, feat_latent, fw_W, fw_b,
           user_weight, item_weight):
    users = x[:, 0].astype(jnp.int32)
    items = x[:, 1].astype(jnp.int32)
    feats = x[:, 2:]
    u_e, i_e, uw, iw = _sc_gather(user_latent, item_latent,
                                  user_weight, item_weight, users, items)
    return _tc_combine(feats, u_e, i_e, uw, iw,
                       feat_latent, fw_W, jnp.reshape(fw_b, (1, 1)))
